# Initial kernel scaffold; baseline (speedup 1.0000x reference)
import jax
import jax.numpy as jnp
from jax.experimental import pallas as pl


def kernel(center_words, target_words, outer_words, V, U):
    raise NotImplementedError("write your pallas kernel here")


# speedup vs baseline: 3.9157x; 3.9157x over previous
import functools

import jax
import jax.numpy as jnp
from jax import lax
from jax.experimental import pallas as pl
from jax.experimental.pallas import tpu as pltpu
from jax.experimental.pallas import tpu_sc as plsc

_B = 16384
_K = 20
_D = 64
_NC = 2
_NS = 16
_NW = _NC * _NS
_CB = _B // _NW
_C = 32
_NCHUNK = _CB // _C
_IDXCHUNK = 128


def _sc_body(cw_hbm, tw_hbm, ow_hbm, v_hbm, u_hbm, score_hbm, sumexp_hbm,
             idx_c, idx_t, idx_o, vrows, trows, orows,
             score_buf, sumexp_buf, sem):
    wid = lax.axis_index("s") * _NC + lax.axis_index("c")
    wbase = wid * _CB

    for i in range(_NCHUNK):
        base = wbase + i * _C
        pltpu.sync_copy(cw_hbm.at[pl.ds(base, _C)], idx_c)
        pltpu.sync_copy(tw_hbm.at[pl.ds(base, _C)], idx_t)
        pltpu.sync_copy(ow_hbm.at[pl.ds(base * _K, _C * _K)], idx_o)
        copies = [
            pltpu.async_copy(v_hbm.at[idx_c], vrows, sem),
            pltpu.async_copy(u_hbm.at[idx_t], trows, sem),
        ]
        for g in range(_C * _K // _IDXCHUNK):
            copies.append(pltpu.async_copy(
                u_hbm.at[idx_o.at[pl.ds(g * _IDXCHUNK, _IDXCHUNK)]],
                orows.at[pl.ds(g * _IDXCHUNK, _IDXCHUNK)], sem))
        for cp in copies:
            cp.wait()

        for g in range(_C // 16):
            lanes = lax.iota(jnp.int32, 16)
            rows = g * 16 + lanes
            orow0 = rows * _K
            zero = jnp.zeros((16,), jnp.float32)

            def dbody(d, carry):
                dcol = jnp.full((16,), d, jnp.int32)
                c_d = plsc.load_gather(vrows, [rows, dcol])
                t_d = plsc.load_gather(trows, [rows, dcol])
                acc_t = carry[0] + t_d * c_d
                accs = []
                for k in range(_K):
                    o_d = plsc.load_gather(orows, [orow0 + k, dcol])
                    accs.append(carry[1 + k] + o_d * c_d)
                return (acc_t, *accs)

            out = lax.fori_loop(0, _D, dbody, (zero,) * (_K + 1))
            score = out[0]
            sumexp = jnp.exp(out[1])
            for k in range(2, _K + 1):
                sumexp = sumexp + jnp.exp(out[k])
            off = i * _C + g * 16
            score_buf[pl.ds(off, 16)] = score
            sumexp_buf[pl.ds(off, 16)] = sumexp

    pltpu.sync_copy(score_buf, score_hbm.at[pl.ds(wbase, _CB)])
    pltpu.sync_copy(sumexp_buf, sumexp_hbm.at[pl.ds(wbase, _CB)])


def _nll_body(score_ref, sumexp_ref, o_ref):
    s = score_ref[...]
    z = sumexp_ref[...]
    o_ref[0, 0] = (jnp.sum(jnp.log(z)) - jnp.sum(s)) / _B


def kernel(center_words, target_words, outer_words, V, U):
    cw = center_words.reshape(_B)
    tw = target_words.reshape(_B)
    ow = outer_words.reshape(_B * _K)

    mesh = plsc.VectorSubcoreMesh(core_axis_name="c", subcore_axis_name="s")
    sc = functools.partial(
        pl.kernel, mesh=mesh,
        compiler_params=pltpu.CompilerParams(
            use_tc_tiling_on_sc=False, needs_layout_passes=False),
        out_type=[jax.ShapeDtypeStruct((_B,), jnp.float32),
                  jax.ShapeDtypeStruct((_B,), jnp.float32)],
        scratch_types=[
            pltpu.VMEM((_C,), jnp.int32),
            pltpu.VMEM((_C,), jnp.int32),
            pltpu.VMEM((_C * _K,), jnp.int32),
            pltpu.VMEM((_C, _D), jnp.float32),
            pltpu.VMEM((_C, _D), jnp.float32),
            pltpu.VMEM((_C * _K, _D), jnp.float32),
            pltpu.VMEM((_CB,), jnp.float32),
            pltpu.VMEM((_CB,), jnp.float32),
            pltpu.SemaphoreType.DMA,
        ],
    )(_sc_body)
    score, sumexp = sc(cw, tw, ow, V, U)

    out = pl.pallas_call(
        _nll_body,
        out_shape=jax.ShapeDtypeStruct((1, 1), jnp.float32),
        out_specs=pl.BlockSpec(memory_space=pltpu.SMEM),
    )(score.reshape(128, 128), sumexp.reshape(128, 128))
    return out[0, 0]


# speedup vs baseline: 4.6213x; 1.1802x over previous
import functools

import jax
import jax.numpy as jnp
from jax import lax
from jax.experimental import pallas as pl
from jax.experimental.pallas import tpu as pltpu
from jax.experimental.pallas import tpu_sc as plsc

_B = 16384
_K = 20
_D = 64
_NC = 2
_NS = 16
_NW = _NC * _NS
_CB = _B // _NW
_C = 32
_NCHUNK = _CB // _C
_IDXCHUNK = 128
_TR = 500000


def _sc_body(cw_hbm, tw_hbm, ow_hbm, v_hbm, u_hbm, score_hbm, sumexp_hbm,
             idx_c, idx_t, idx_o, idx_c2, idx_t2, idx_o2,
             vrows, trows, orows, norm_buf, score_buf, sumexp_buf, sem):
    wid = lax.axis_index("s") * _NC + lax.axis_index("c")
    wbase = wid * _CB
    lanes = lax.iota(jnp.int32, 16)
    last = lanes == 15

    def chunk_body(i, _):
        base = wbase + i * _C
        pltpu.sync_copy(cw_hbm.at[pl.ds(base, _C)], idx_c.at[pl.ds(0, _C)])
        pltpu.sync_copy(tw_hbm.at[pl.ds(base, _C)], idx_t.at[pl.ds(0, _C)])
        pltpu.sync_copy(ow_hbm.at[pl.ds(base * _K, _C * _K)],
                        idx_o.at[pl.ds(0, _C * _K)])
        for j in range(_C // 16):
            idx_c2[pl.ds(j * 16, 16)] = idx_c[pl.ds(j * 16, 16)] >> 1
            idx_t2[pl.ds(j * 16, 16)] = idx_t[pl.ds(j * 16, 16)] >> 1
        for j in range(_C * _K // 16):
            idx_o2[pl.ds(j * 16, 16)] = idx_o[pl.ds(j * 16, 16)] >> 1
        copies = [
            pltpu.async_copy(v_hbm.at[idx_c2], vrows, sem),
            pltpu.async_copy(u_hbm.at[idx_t2], trows, sem),
        ]
        for g in range(_C * _K // _IDXCHUNK):
            copies.append(pltpu.async_copy(
                u_hbm.at[idx_o2.at[pl.ds(g * _IDXCHUNK, _IDXCHUNK)]],
                orows.at[pl.ds(g * _IDXCHUNK, _IDXCHUNK)], sem))
        for cp in copies:
            cp.wait()

        def ebody(e, _):
            cvec = idx_c[pl.ds(e, 16)]
            tvec = idx_t[pl.ds(e, 16)]
            ovec0 = (idx_o[pl.ds(e * _K, 16)] & 1) * _D
            ovec1 = (idx_o[pl.ds(e * _K + 16, 16)] & 1) * _D
            pc = (cvec[0] & 1) * _D
            pt = (tvec[0] & 1) * _D
            c = [vrows[e, pl.ds(pc + 16 * j, 16)] for j in range(4)]
            prod = c[0] * trows[e, pl.ds(pt, 16)]
            for j in range(1, 4):
                prod = prod + c[j] * trows[e, pl.ds(pt + 16 * j, 16)]
            plsc.store_scatter(score_buf,
                               [jnp.full((16,), i * _C + e, jnp.int32)],
                               plsc.cumsum(prod), mask=last)
            for k in range(_K):
                p = ovec0[k] if k < 16 else ovec1[k - 16]
                row = e * _K + k
                acc = c[0] * orows[row, pl.ds(p, 16)]
                for j in range(1, 4):
                    acc = acc + c[j] * orows[row, pl.ds(p + 16 * j, 16)]
                plsc.store_scatter(norm_buf,
                                   [jnp.full((16,), k * _C + e, jnp.int32)],
                                   plsc.cumsum(acc), mask=last)
            return 0

        lax.fori_loop(0, _C, ebody, 0)

        for g in range(_C // 16):
            s = jnp.exp(norm_buf[pl.ds(g * 16, 16)])
            for k in range(1, _K):
                s = s + jnp.exp(norm_buf[pl.ds(k * _C + g * 16, 16)])
            sumexp_buf[pl.ds(i * _C + g * 16, 16)] = s
        return 0

    lax.fori_loop(0, _NCHUNK, chunk_body, 0)

    pltpu.sync_copy(score_buf, score_hbm.at[pl.ds(wbase, _CB)])
    pltpu.sync_copy(sumexp_buf, sumexp_hbm.at[pl.ds(wbase, _CB)])


def _nll_body(score_ref, sumexp_ref, o_ref):
    s = score_ref[...]
    z = sumexp_ref[...]
    o_ref[0, 0] = (jnp.sum(jnp.log(z)) - jnp.sum(s)) / _B


def kernel(center_words, target_words, outer_words, V, U):
    cw = center_words.reshape(_B)
    tw = target_words.reshape(_B)
    ow = outer_words.reshape(_B * _K)
    v2 = V.reshape(_TR, 2 * _D)
    u2 = U.reshape(_TR, 2 * _D)

    mesh = plsc.VectorSubcoreMesh(core_axis_name="c", subcore_axis_name="s")
    sc = functools.partial(
        pl.kernel, mesh=mesh,
        compiler_params=pltpu.CompilerParams(
            use_tc_tiling_on_sc=False, needs_layout_passes=False),
        out_type=[jax.ShapeDtypeStruct((_B,), jnp.float32),
                  jax.ShapeDtypeStruct((_B,), jnp.float32)],
        scratch_types=[
            pltpu.VMEM((_C + 16,), jnp.int32),
            pltpu.VMEM((_C + 16,), jnp.int32),
            pltpu.VMEM((_C * _K + 16,), jnp.int32),
            pltpu.VMEM((_C,), jnp.int32),
            pltpu.VMEM((_C,), jnp.int32),
            pltpu.VMEM((_C * _K,), jnp.int32),
            pltpu.VMEM((_C, 2 * _D), jnp.float32),
            pltpu.VMEM((_C, 2 * _D), jnp.float32),
            pltpu.VMEM((_C * _K, 2 * _D), jnp.float32),
            pltpu.VMEM((_C * _K,), jnp.float32),
            pltpu.VMEM((_CB,), jnp.float32),
            pltpu.VMEM((_CB,), jnp.float32),
            pltpu.SemaphoreType.DMA,
        ],
    )(_sc_body)
    score, sumexp = sc(cw, tw, ow, v2, u2)

    out = pl.pallas_call(
        _nll_body,
        out_shape=jax.ShapeDtypeStruct((1, 1), jnp.float32),
        out_specs=pl.BlockSpec(memory_space=pltpu.SMEM),
    )(score.reshape(128, 128), sumexp.reshape(128, 128))
    return out[0, 0]
